# baseline (speedup 1.0000x reference)
import jax
import jax.numpy as jnp
from jax import lax
from jax.experimental import pallas as pl
from jax.experimental.pallas import tpu as pltpu
from jax.experimental.pallas import tpu_sc as plsc

NC = 2
NS = 16
NW = NC * NS
L = 16
BATCH = 16384
DIM = 16
NROWS = 1000000
GRP = 8
GW = GRP * DIM
BPW = BATCH // NW
NCHUNK = 4
CHUNK = BPW // NCHUNK


def _body(u_hbm, i_hbm, ut_hbm, it_hbm, out_hbm,
          idx_u, idx_i, gid_u, gid_i, ue, ie, out_v, sem):
    wid = lax.axis_index("s") * NC + lax.axis_index("c")
    base = wid * BPW

    pltpu.sync_copy(u_hbm.at[wid], idx_u)
    pltpu.sync_copy(i_hbm.at[wid], idx_i)

    for j in range(NCHUNK):
        for t in range(CHUNK // L):
            sl = pl.ds(t * L, L)
            gid_u.at[j][sl] = lax.shift_right_logical(idx_u.at[j][sl], 3)
            gid_i.at[j][sl] = lax.shift_right_logical(idx_i.at[j][sl], 3)

    def compute_chunk(j, buf):
        def group(g, carry):
            r0 = g * L
            rows = lax.iota(jnp.int32, L) + r0
            su = lax.shift_left(jnp.bitwise_and(idx_u.at[j][pl.ds(r0, L)], 7), 4)
            si = lax.shift_left(jnp.bitwise_and(idx_i.at[j][pl.ds(r0, L)], 7), 4)
            acc = jnp.zeros((L,), jnp.float32)
            for d in range(DIM):
                uc = plsc.load_gather(ue.at[buf], [rows, su + d])
                ic = plsc.load_gather(ie.at[buf], [rows, si + d])
                acc = acc + uc * ic
            out_v[pl.ds(j * CHUNK + r0, L)] = acc
            return carry

        lax.fori_loop(0, CHUNK // L, group, 0)

    def fire(j, buf):
        return (pltpu.async_copy(ut_hbm.at[gid_u.at[j]], ue.at[buf], sem),
                pltpu.async_copy(it_hbm.at[gid_i.at[j]], ie.at[buf], sem))

    pending = fire(0, 0)
    for j in range(NCHUNK):
        for c in pending:
            c.wait()
        if j + 1 < NCHUNK:
            nxt = fire(j + 1, (j + 1) % 2)
        compute_chunk(j, j % 2)
        if j + 1 < NCHUNK:
            pending = nxt

    pltpu.sync_copy(out_v, out_hbm.at[pl.ds(base, BPW)])


@jax.jit
def kernel(u, i, user_table, item_table):
    u32 = u.astype(jnp.int32)
    i32 = i.astype(jnp.int32)
    u3 = u32.reshape(NW, NCHUNK, CHUNK)
    i3 = i32.reshape(NW, NCHUNK, CHUNK)
    one = (1 - lax.shift_right_logical(jnp.min(u32), 30)).astype(jnp.float32)
    ut2 = user_table[:NROWS].reshape(NROWS // GRP, GW) * one
    it2 = item_table[:NROWS].reshape(NROWS // GRP, GW)
    mesh = plsc.VectorSubcoreMesh(core_axis_name="c", subcore_axis_name="s")
    f = pl.kernel(
        _body,
        out_type=jax.ShapeDtypeStruct((BATCH,), jnp.float32),
        mesh=mesh,
        compiler_params=pltpu.CompilerParams(needs_layout_passes=False),
        scratch_types=[
            pltpu.VMEM((NCHUNK, CHUNK), jnp.int32),
            pltpu.VMEM((NCHUNK, CHUNK), jnp.int32),
            pltpu.VMEM((NCHUNK, CHUNK), jnp.int32),
            pltpu.VMEM((NCHUNK, CHUNK), jnp.int32),
            pltpu.VMEM((2, CHUNK, GW), jnp.float32),
            pltpu.VMEM((2, CHUNK, GW), jnp.float32),
            pltpu.VMEM((BPW,), jnp.float32),
            pltpu.SemaphoreType.DMA,
        ],
    )
    return f(u3, i3, ut2, it2)


# speedup vs baseline: 1.5718x; 1.5718x over previous
import jax
import jax.numpy as jnp
from jax import lax
from jax.experimental import pallas as pl
from jax.experimental.pallas import tpu as pltpu
from jax.experimental.pallas import tpu_sc as plsc

NC = 2
NS = 16
NW = NC * NS
L = 16
BATCH = 16384
DIM = 16
BPW = BATCH // NW
NCHUNK = 4
CHUNK = BPW // NCHUNK


def _body(u_hbm, i_hbm, ut_hbm, it_hbm, out_hbm,
          idx_u, idx_i, ue, ie, out_v, *sems):
    wid = lax.axis_index("s") * NC + lax.axis_index("c")
    base = wid * BPW

    pltpu.sync_copy(u_hbm.at[wid], idx_u)
    pltpu.sync_copy(i_hbm.at[wid], idx_i)

    def fire_chunk(j, buf):
        def enq(g, carry):
            r0 = g * L
            iu_vec = idx_u[pl.ds(j * CHUNK + r0, L)]
            ii_vec = idx_i[pl.ds(j * CHUNK + r0, L)]
            for l in range(L):
                pltpu.async_copy(ut_hbm.at[iu_vec[l]],
                                 ue.at[buf, r0 + l], sems[0])
                pltpu.async_copy(it_hbm.at[ii_vec[l]],
                                 ie.at[buf, r0 + l], sems[1])
            return carry

        lax.fori_loop(0, CHUNK // L, enq, 0)

    def drain_chunk(buf):
        pltpu.make_async_copy(ut_hbm.at[pl.ds(0, CHUNK)],
                              ue.at[buf], sems[0]).wait()
        pltpu.make_async_copy(it_hbm.at[pl.ds(0, CHUNK)],
                              ie.at[buf], sems[1]).wait()

    def compute_chunk(j, buf):
        def group(g, carry):
            r0 = g * L
            rows = lax.iota(jnp.int32, L) + r0
            acc = jnp.zeros((L,), jnp.float32)
            for d in range(DIM):
                col = jnp.full((L,), d, jnp.int32)
                uc = plsc.load_gather(ue.at[buf], [rows, col])
                ic = plsc.load_gather(ie.at[buf], [rows, col])
                acc = acc + uc * ic
            out_v[pl.ds(j * CHUNK + r0, L)] = acc
            return carry

        lax.fori_loop(0, CHUNK // L, group, 0)

    fire_chunk(0, 0)
    for j in range(NCHUNK):
        drain_chunk(j % 2)
        if j + 1 < NCHUNK:
            fire_chunk(j + 1, (j + 1) % 2)
        compute_chunk(j, j % 2)

    pltpu.sync_copy(out_v, out_hbm.at[pl.ds(base, BPW)])


@jax.jit
def kernel(u, i, user_table, item_table):
    u2 = u.astype(jnp.int32).reshape(NW, BPW)
    i2 = i.astype(jnp.int32).reshape(NW, BPW)
    mesh = plsc.VectorSubcoreMesh(core_axis_name="c", subcore_axis_name="s")
    f = pl.kernel(
        _body,
        out_type=jax.ShapeDtypeStruct((BATCH,), jnp.float32),
        mesh=mesh,
        compiler_params=pltpu.CompilerParams(needs_layout_passes=False),
        scratch_types=[
            pltpu.VMEM((BPW,), jnp.int32),
            pltpu.VMEM((BPW,), jnp.int32),
            pltpu.VMEM((2, CHUNK, DIM), jnp.float32),
            pltpu.VMEM((2, CHUNK, DIM), jnp.float32),
            pltpu.VMEM((BPW,), jnp.float32),
        ] + [pltpu.SemaphoreType.DMA] * 2,
    )
    return f(u2, i2, user_table, item_table)
